# Initial kernel scaffold; baseline (speedup 1.0000x reference)
import jax
import jax.numpy as jnp
from jax.experimental import pallas as pl


def kernel(edges, entity_embed_init, W_rel1, W_self1, b1, W_rel2, W_self2, b2):
    raise NotImplementedError("write your pallas kernel here")


# speedup vs baseline: 1.8718x; 1.8718x over previous
import functools

import jax
import jax.numpy as jnp
from jax import lax
from jax.experimental import pallas as pl
from jax.experimental.pallas import tpu as pltpu
from jax.experimental.pallas import tpu_sc as plsc

f32 = jnp.float32
i32 = jnp.int32

NC = 2
NS = 16
L = 16

CHUNK = 50
BN = 1000
ZROWS = 40


def _mesh():
    return plsc.VectorSubcoreMesh(core_axis_name="c", subcore_axis_name="s")


def _tc_transform(h, W_rel):
    N, D = h.shape
    R = W_rel.shape[0]
    H = D // 2
    NBN = N // BN

    def body(h_ref, w_ref, outl_ref, outr_ref):
        res = jnp.dot(h_ref[...], w_ref[0], preferred_element_type=f32)
        outl_ref[...] = res[:, :H]
        outr_ref[...] = res[:, H:]

    return pl.pallas_call(
        body,
        grid=(NBN, R),
        in_specs=[
            pl.BlockSpec((BN, D), lambda n, r: (n, 0)),
            pl.BlockSpec((1, D, D), lambda n, r: (r, 0, 0)),
        ],
        out_specs=[
            pl.BlockSpec((BN, H), lambda n, r: (r * NBN + n, 0)),
            pl.BlockSpec((BN, H), lambda n, r: (r * NBN + n, 0)),
        ],
        out_shape=[
            jax.ShapeDtypeStruct((R * N, H), f32),
            jax.ShapeDtypeStruct((R * N, H), f32),
        ],
    )(h, W_rel)


def _sc_msg(tl, tr, gidx2, dst2, N, H, deg_args=None):
    with_deg = deg_args is not None
    NCHUNKS = dst2.shape[0]
    TPR = NCHUNKS // NS
    NIO = 10
    NPT = N // NIO
    TPRH = TPR // 5
    DR = 1280
    DPT = DR // NIO

    out_type = [jax.ShapeDtypeStruct((NC, N, H), f32)]
    scratch = [
        pltpu.VMEM((TPRH, CHUNK), i32),
        pltpu.VMEM((TPRH, CHUNK), i32),
        pltpu.VMEM((CHUNK, H), f32),
        pltpu.SemaphoreType.DMA,
        pltpu.SemaphoreType.DMA,
        pltpu.VMEM_SHARED((N, H), f32),
    ]
    if with_deg:
        out_type.append(jax.ShapeDtypeStruct((DR, H), f32))
        scratch += [
            pltpu.VMEM((TPRH, CHUNK), i32),
            pltpu.VMEM((TPRH, CHUNK), i32),
            pltpu.VMEM((CHUNK, H), f32),
            pltpu.SemaphoreType.DMA,
            pltpu.VMEM_SHARED((DR, H), f32),
        ]

    @functools.partial(
        pl.kernel,
        out_type=out_type,
        mesh=_mesh(),
        scratch_types=scratch,
    )
    def msg_kernel(*refs):
        if with_deg:
            (tl_hbm, tr_hbm, gidx_hbm, dst_hbm, pat_hbm, d3_hbm, d7_hbm,
             acc_hbm, deg_hbm, gbuf, dbuf, rows, gsem, ssem, shared,
             d3buf, pbuf, deg_rows, dsem, shared_deg) = refs
        else:
            (tl_hbm, tr_hbm, gidx_hbm, dst_hbm, acc_hbm,
             gbuf, dbuf, rows, gsem, ssem, shared) = refs
        c = lax.axis_index("c")
        s = lax.axis_index("s")

        def zfill(k, _):
            rows[k // (H // L), pl.ds((k % (H // L)) * L, L)] = \
                jnp.zeros((L,), f32)
            return 0

        lax.fori_loop(0, CHUNK * (H // L), zfill, 0)

        @pl.when(s < NIO)
        def _():
            def zcopy(k, _):
                pltpu.sync_copy(rows.at[pl.ds(0, ZROWS)],
                                shared.at[pl.ds(s * NPT + k * ZROWS, ZROWS)])
                return 0

            lax.fori_loop(0, NPT // ZROWS, zcopy, 0)

            if with_deg:
                @pl.when(c == 0)
                def _():
                    def zcopy2(k, _):
                        pltpu.sync_copy(
                            rows.at[pl.ds(0, 8)],
                            shared_deg.at[pl.ds(s * DPT + k * 8, 8)])
                        return 0

                    lax.fori_loop(0, DPT // 8, zcopy2, 0)

        plsc.subcore_barrier()

        def run(t_ref, do_deg):
            def body(i, _):
                pltpu.async_copy(t_ref.at[gbuf.at[i]], rows, gsem).wait()
                pltpu.async_copy(rows, shared.at[dbuf.at[i]], ssem,
                                 add=True).wait()
                if do_deg:
                    pltpu.async_copy(pat_hbm.at[pbuf.at[i]], deg_rows,
                                     dsem).wait()
                    pltpu.async_copy(deg_rows, shared_deg.at[d3buf.at[i]],
                                     dsem, add=True).wait()
                return 0

            for p in range(TPR // TPRH):
                pltpu.sync_copy(
                    gidx_hbm.at[pl.ds(s * TPR + p * TPRH, TPRH)], gbuf)
                pltpu.sync_copy(
                    dst_hbm.at[pl.ds(s * TPR + p * TPRH, TPRH)], dbuf)
                if do_deg:
                    pltpu.sync_copy(
                        d3_hbm.at[pl.ds(s * TPR + p * TPRH, TPRH)], d3buf)
                    pltpu.sync_copy(
                        d7_hbm.at[pl.ds(s * TPR + p * TPRH, TPRH)], pbuf)
                lax.fori_loop(0, TPRH, body, 0)

        @pl.when(c == 0)
        def _():
            run(tl_hbm, with_deg)

        @pl.when(c == 1)
        def _():
            run(tr_hbm, False)

        plsc.subcore_barrier()

        @pl.when(s < NIO)
        def _():
            pltpu.sync_copy(shared.at[pl.ds(s * NPT, NPT)],
                            acc_hbm.at[c, pl.ds(s * NPT, NPT)])

            if with_deg:
                @pl.when(c == 0)
                def _():
                    pltpu.sync_copy(shared_deg.at[pl.ds(s * DPT, DPT)],
                                    deg_hbm.at[pl.ds(s * DPT, DPT)])

    if with_deg:
        pat, d3_2, d7_2 = deg_args
        return msg_kernel(tl, tr, gidx2, dst2, pat, d3_2, d7_2)
    return msg_kernel(tl, tr, gidx2, dst2)


def _tc_finish_transform(acc, deg3, h_in, W_self, b2d, W_rel_next, relu):
    N, D = h_in.shape
    R = W_rel_next.shape[0]
    H = D // 2
    NBN = N // BN

    def body(acc_ref, deg_ref, h_ref, ws_ref, b_ref, wr_ref,
             hout_ref, tl_ref, tr_ref, hscr):
        r = pl.program_id(1)

        @pl.when(r == 0)
        def _():
            inv = 1.0 / jnp.clip(deg_ref[...], 1.0, None)
            msg = jnp.concatenate([acc_ref[0] * inv, acc_ref[1] * inv], axis=1)
            base = jnp.dot(h_ref[...], ws_ref[...],
                           preferred_element_type=f32) + b_ref[...]
            hn = msg + base
            if relu:
                hn = jnp.maximum(hn, 0.0)
            hscr[...] = hn
            hout_ref[...] = hn

        res = jnp.dot(hscr[...], wr_ref[0], preferred_element_type=f32)
        tl_ref[...] = res[:, :H]
        tr_ref[...] = res[:, H:]

    return pl.pallas_call(
        body,
        grid=(NBN, R),
        in_specs=[
            pl.BlockSpec((NC, BN, H), lambda n, r: (0, n, 0)),
            pl.BlockSpec((BN, 1), lambda n, r: (n, 0)),
            pl.BlockSpec((BN, D), lambda n, r: (n, 0)),
            pl.BlockSpec((D, D), lambda n, r: (0, 0)),
            pl.BlockSpec((1, D), lambda n, r: (0, 0)),
            pl.BlockSpec((1, D, D), lambda n, r: (r, 0, 0)),
        ],
        out_specs=[
            pl.BlockSpec((BN, D), lambda n, r: (n, 0)),
            pl.BlockSpec((BN, H), lambda n, r: (r * NBN + n, 0)),
            pl.BlockSpec((BN, H), lambda n, r: (r * NBN + n, 0)),
        ],
        out_shape=[
            jax.ShapeDtypeStruct((N, D), f32),
            jax.ShapeDtypeStruct((R * N, H), f32),
            jax.ShapeDtypeStruct((R * N, H), f32),
        ],
        scratch_shapes=[pltpu.VMEM((BN, D), f32)],
    )(acc, deg3, h_in, W_self, b2d, W_rel_next)


def _tc_finish(acc, deg3, h_in, W_self, b2d):
    N, D = h_in.shape
    H = D // 2
    NBN = N // BN

    def body(acc_ref, deg_ref, h_ref, ws_ref, b_ref, out_ref):
        inv = 1.0 / jnp.clip(deg_ref[...], 1.0, None)
        msg = jnp.concatenate([acc_ref[0] * inv, acc_ref[1] * inv], axis=1)
        base = jnp.dot(h_ref[...], ws_ref[...],
                       preferred_element_type=f32) + b_ref[...]
        out_ref[...] = msg + base

    return pl.pallas_call(
        body,
        grid=(NBN,),
        in_specs=[
            pl.BlockSpec((NC, BN, H), lambda n: (0, n, 0)),
            pl.BlockSpec((BN, 1), lambda n: (n, 0)),
            pl.BlockSpec((BN, D), lambda n: (n, 0)),
            pl.BlockSpec((D, D), lambda n: (0, 0)),
            pl.BlockSpec((1, D), lambda n: (0, 0)),
        ],
        out_specs=pl.BlockSpec((BN, D), lambda n: (n, 0)),
        out_shape=jax.ShapeDtypeStruct((N, D), f32),
    )(acc, deg3, h_in, W_self, b2d)


def kernel(edges, entity_embed_init, W_rel1, W_self1, b1, W_rel2, W_self2, b2):
    h = entity_embed_init
    N, D = h.shape
    R = W_rel1.shape[0]
    H = D // 2
    b1_2 = b1.reshape(1, D)
    b2_2 = b2.reshape(1, D)

    for i in range(edges.shape[0]):
        src = edges[i, 0].astype(i32)
        rel = (edges[i, 1] % R).astype(i32)
        dst = edges[i, 2].astype(i32)

        gidx2 = (rel * N + src).reshape(-1, CHUNK)
        dst2 = dst.reshape(-1, CHUNK)
        d3_2 = (dst >> 3).reshape(-1, CHUNK)
        d7_2 = (dst & 7).reshape(-1, CHUNK)
        pat = (jnp.arange(H, dtype=i32)[None, :] // L
               == jnp.arange(8, dtype=i32)[:, None]).astype(f32)

        tl1, tr1 = _tc_transform(h, W_rel1)
        acc1, deg = _sc_msg(tl1, tr1, gidx2, dst2, N, H,
                            deg_args=(pat, d3_2, d7_2))
        deg3 = deg.reshape(-1, 8, L)[:, :, 0].reshape(-1)[:N].reshape(N, 1)
        h1, tl2, tr2 = _tc_finish_transform(acc1, deg3, h, W_self1, b1_2,
                                            W_rel2, relu=True)
        acc2, = _sc_msg(tl2, tr2, gidx2, dst2, N, H)
        h = _tc_finish(acc2, deg3, h1, W_self2, b2_2)
    return h


# speedup vs baseline: 1.9161x; 1.0237x over previous
import functools

import jax
import jax.numpy as jnp
from jax import lax
from jax.experimental import pallas as pl
from jax.experimental.pallas import tpu as pltpu
from jax.experimental.pallas import tpu_sc as plsc

f32 = jnp.float32
i32 = jnp.int32

NC = 2
NS = 16
L = 16

CHUNK = 125
BN = 1000
ZROWS = 40


def _mesh():
    return plsc.VectorSubcoreMesh(core_axis_name="c", subcore_axis_name="s")


def _tc_transform(h, W_rel):
    N, D = h.shape
    R = W_rel.shape[0]
    H = D // 2
    NBN = N // BN

    def body(h_ref, w_ref, outl_ref, outr_ref):
        res = jnp.dot(h_ref[...], w_ref[0], preferred_element_type=f32)
        outl_ref[...] = res[:, :H]
        outr_ref[...] = res[:, H:]

    return pl.pallas_call(
        body,
        grid=(NBN, R),
        in_specs=[
            pl.BlockSpec((BN, D), lambda n, r: (n, 0)),
            pl.BlockSpec((1, D, D), lambda n, r: (r, 0, 0)),
        ],
        out_specs=[
            pl.BlockSpec((BN, H), lambda n, r: (r * NBN + n, 0)),
            pl.BlockSpec((BN, H), lambda n, r: (r * NBN + n, 0)),
        ],
        out_shape=[
            jax.ShapeDtypeStruct((R * N, H), f32),
            jax.ShapeDtypeStruct((R * N, H), f32),
        ],
    )(h, W_rel)


def _sc_msg(tl, tr, gidx2, dst2, N, H, deg_args=None):
    with_deg = deg_args is not None
    NCHUNKS = dst2.shape[0]
    TPR = NCHUNKS // NS
    NIO = 10
    NPT = N // NIO
    TPRH = TPR // 2
    DR = 1280
    DPT = DR // NIO
    D0 = 24

    out_type = [jax.ShapeDtypeStruct((NC, N, H), f32)]
    scratch = [
        pltpu.VMEM((TPRH, CHUNK), i32),
        pltpu.VMEM((TPRH, CHUNK), i32),
        pltpu.VMEM((CHUNK, H), f32),
        pltpu.SemaphoreType.DMA,
        pltpu.SemaphoreType.DMA,
        pltpu.VMEM_SHARED((N, H), f32),
    ]
    if with_deg:
        out_type.append(jax.ShapeDtypeStruct((NC, DR, H), f32))
        scratch += [pltpu.VMEM_SHARED((DR, H), f32)]

    @functools.partial(
        pl.kernel,
        out_type=out_type,
        mesh=_mesh(),
        scratch_types=scratch,
    )
    def msg_kernel(*refs):
        if with_deg:
            (tl_hbm, tr_hbm, gidx_hbm, dst_hbm, pat_hbm, d3_hbm, d7_hbm,
             acc_hbm, deg_hbm, gbuf, dbuf, rows, gsem, ssem, shared,
             shared_deg) = refs
        else:
            (tl_hbm, tr_hbm, gidx_hbm, dst_hbm, acc_hbm,
             gbuf, dbuf, rows, gsem, ssem, shared) = refs
        c = lax.axis_index("c")
        s = lax.axis_index("s")

        def zfill(k, _):
            rows[k // (H // L), pl.ds((k % (H // L)) * L, L)] = \
                jnp.zeros((L,), f32)
            return 0

        lax.fori_loop(0, CHUNK * (H // L), zfill, 0)

        @pl.when(s < NIO)
        def _():
            def zcopy(k, _):
                pltpu.sync_copy(rows.at[pl.ds(0, ZROWS)],
                                shared.at[pl.ds(s * NPT + k * ZROWS, ZROWS)])
                return 0

            lax.fori_loop(0, NPT // ZROWS, zcopy, 0)

            if with_deg:
                def zcopy2(k, _):
                    pltpu.sync_copy(rows.at[pl.ds(0, 8)],
                                    shared_deg.at[pl.ds(s * DPT + k * 8, 8)])
                    return 0

                lax.fori_loop(0, DPT // 8, zcopy2, 0)

        plsc.subcore_barrier()

        def run(t_ref):
            def body(i, _):
                pltpu.async_copy(t_ref.at[gbuf.at[i]], rows, gsem).wait()
                pltpu.async_copy(rows, shared.at[dbuf.at[i]], ssem,
                                 add=True).wait()
                return 0

            for p in range(TPR // TPRH):
                pltpu.sync_copy(
                    gidx_hbm.at[pl.ds(s * TPR + p * TPRH, TPRH)], gbuf)
                pltpu.sync_copy(
                    dst_hbm.at[pl.ds(s * TPR + p * TPRH, TPRH)], dbuf)
                lax.fori_loop(0, TPRH, body, 0)

        @pl.when(c == 0)
        def _():
            run(tl_hbm)

        @pl.when(c == 1)
        def _():
            run(tr_hbm)

        if with_deg:
            def deg_run(off, cnt):
                def dbody(i, _):
                    pltpu.async_copy(pat_hbm.at[gbuf.at[i]], rows,
                                     gsem).wait()
                    pltpu.async_copy(rows, shared_deg.at[dbuf.at[i]], ssem,
                                     add=True).wait()
                    return 0

                for p in range(TPR // TPRH):
                    base = s * TPR + p * TPRH + off
                    pltpu.sync_copy(d7_hbm.at[pl.ds(base, cnt)],
                                    gbuf.at[pl.ds(0, cnt)])
                    pltpu.sync_copy(d3_hbm.at[pl.ds(base, cnt)],
                                    dbuf.at[pl.ds(0, cnt)])
                    lax.fori_loop(0, cnt, dbody, 0)

            @pl.when(c == 0)
            def _():
                deg_run(0, D0)

            @pl.when(c == 1)
            def _():
                deg_run(D0, TPRH - D0)

        plsc.subcore_barrier()

        @pl.when(s < NIO)
        def _():
            pltpu.sync_copy(shared.at[pl.ds(s * NPT, NPT)],
                            acc_hbm.at[c, pl.ds(s * NPT, NPT)])

            if with_deg:
                pltpu.sync_copy(shared_deg.at[pl.ds(s * DPT, DPT)],
                                deg_hbm.at[c, pl.ds(s * DPT, DPT)])

    if with_deg:
        pat, d3_2, d7_2 = deg_args
        return msg_kernel(tl, tr, gidx2, dst2, pat, d3_2, d7_2)
    return msg_kernel(tl, tr, gidx2, dst2)


def _tc_finish_transform(acc, deg3, h_in, W_self, b2d, W_rel_next, relu):
    N, D = h_in.shape
    R = W_rel_next.shape[0]
    H = D // 2
    NBN = N // BN

    def body(acc_ref, deg_ref, h_ref, ws_ref, b_ref, wr_ref,
             hout_ref, tl_ref, tr_ref, hscr):
        r = pl.program_id(1)

        @pl.when(r == 0)
        def _():
            inv = 1.0 / jnp.clip(deg_ref[0] + deg_ref[1], 1.0, None)
            msg = jnp.concatenate([acc_ref[0] * inv, acc_ref[1] * inv], axis=1)
            base = jnp.dot(h_ref[...], ws_ref[...],
                           preferred_element_type=f32) + b_ref[...]
            hn = msg + base
            if relu:
                hn = jnp.maximum(hn, 0.0)
            hscr[...] = hn
            hout_ref[...] = hn

        res = jnp.dot(hscr[...], wr_ref[0], preferred_element_type=f32)
        tl_ref[...] = res[:, :H]
        tr_ref[...] = res[:, H:]

    return pl.pallas_call(
        body,
        grid=(NBN, R),
        in_specs=[
            pl.BlockSpec((NC, BN, H), lambda n, r: (0, n, 0)),
            pl.BlockSpec((NC, BN, 1), lambda n, r: (0, n, 0)),
            pl.BlockSpec((BN, D), lambda n, r: (n, 0)),
            pl.BlockSpec((D, D), lambda n, r: (0, 0)),
            pl.BlockSpec((1, D), lambda n, r: (0, 0)),
            pl.BlockSpec((1, D, D), lambda n, r: (r, 0, 0)),
        ],
        out_specs=[
            pl.BlockSpec((BN, D), lambda n, r: (n, 0)),
            pl.BlockSpec((BN, H), lambda n, r: (r * NBN + n, 0)),
            pl.BlockSpec((BN, H), lambda n, r: (r * NBN + n, 0)),
        ],
        out_shape=[
            jax.ShapeDtypeStruct((N, D), f32),
            jax.ShapeDtypeStruct((R * N, H), f32),
            jax.ShapeDtypeStruct((R * N, H), f32),
        ],
        scratch_shapes=[pltpu.VMEM((BN, D), f32)],
    )(acc, deg3, h_in, W_self, b2d, W_rel_next)


def _tc_finish(acc, deg3, h_in, W_self, b2d):
    N, D = h_in.shape
    H = D // 2
    NBN = N // BN

    def body(acc_ref, deg_ref, h_ref, ws_ref, b_ref, out_ref):
        inv = 1.0 / jnp.clip(deg_ref[0] + deg_ref[1], 1.0, None)
        msg = jnp.concatenate([acc_ref[0] * inv, acc_ref[1] * inv], axis=1)
        base = jnp.dot(h_ref[...], ws_ref[...],
                       preferred_element_type=f32) + b_ref[...]
        out_ref[...] = msg + base

    return pl.pallas_call(
        body,
        grid=(NBN,),
        in_specs=[
            pl.BlockSpec((NC, BN, H), lambda n: (0, n, 0)),
            pl.BlockSpec((NC, BN, 1), lambda n: (0, n, 0)),
            pl.BlockSpec((BN, D), lambda n: (n, 0)),
            pl.BlockSpec((D, D), lambda n: (0, 0)),
            pl.BlockSpec((1, D), lambda n: (0, 0)),
        ],
        out_specs=pl.BlockSpec((BN, D), lambda n: (n, 0)),
        out_shape=jax.ShapeDtypeStruct((N, D), f32),
    )(acc, deg3, h_in, W_self, b2d)


def kernel(edges, entity_embed_init, W_rel1, W_self1, b1, W_rel2, W_self2, b2):
    h = entity_embed_init
    N, D = h.shape
    R = W_rel1.shape[0]
    H = D // 2
    b1_2 = b1.reshape(1, D)
    b2_2 = b2.reshape(1, D)

    for i in range(edges.shape[0]):
        src = edges[i, 0].astype(i32)
        rel = (edges[i, 1] % R).astype(i32)
        dst = edges[i, 2].astype(i32)

        gidx2 = (rel * N + src).reshape(-1, CHUNK)
        dst2 = dst.reshape(-1, CHUNK)
        d3_2 = (dst >> 3).reshape(-1, CHUNK)
        d7_2 = (dst & 7).reshape(-1, CHUNK)
        pat = (jnp.arange(H, dtype=i32)[None, :] // L
               == jnp.arange(8, dtype=i32)[:, None]).astype(f32)

        tl1, tr1 = _tc_transform(h, W_rel1)
        acc1, deg = _sc_msg(tl1, tr1, gidx2, dst2, N, H,
                            deg_args=(pat, d3_2, d7_2))
        deg3 = (deg.reshape(NC, -1, 8, L)[:, :, :, 0]
                .reshape(NC, -1)[:, :N].reshape(NC, N, 1))
        h1, tl2, tr2 = _tc_finish_transform(acc1, deg3, h, W_self1, b1_2,
                                            W_rel2, relu=True)
        acc2, = _sc_msg(tl2, tr2, gidx2, dst2, N, H)
        h = _tc_finish(acc2, deg3, h1, W_self2, b2_2)
    return h


# speedup vs baseline: 3.4017x; 1.7753x over previous
import functools

import jax
import jax.numpy as jnp
from jax import lax
from jax.experimental import pallas as pl
from jax.experimental.pallas import tpu as pltpu
from jax.experimental.pallas import tpu_sc as plsc

f32 = jnp.float32
i32 = jnp.int32

NC = 2
NS = 16
L = 16

CHUNK = 125
BN = 1000
ZROWS = 40


def _mesh():
    return plsc.VectorSubcoreMesh(core_axis_name="c", subcore_axis_name="s")


def _tc_transform(h, W_rel):
    N, D = h.shape
    R = W_rel.shape[0]
    H = D // 2
    NBN = N // BN

    def body(h_ref, w_ref, outl_ref, outr_ref):
        res = jnp.dot(h_ref[...], w_ref[0], preferred_element_type=f32)
        outl_ref[...] = res[:, :H]
        outr_ref[...] = res[:, H:]

    return pl.pallas_call(
        body,
        grid=(NBN, R),
        in_specs=[
            pl.BlockSpec((BN, D), lambda n, r: (n, 0)),
            pl.BlockSpec((1, D, D), lambda n, r: (r, 0, 0)),
        ],
        out_specs=[
            pl.BlockSpec((BN, H), lambda n, r: (r * NBN + n, 0)),
            pl.BlockSpec((BN, H), lambda n, r: (r * NBN + n, 0)),
        ],
        out_shape=[
            jax.ShapeDtypeStruct((R * N, H), f32),
            jax.ShapeDtypeStruct((R * N, H), f32),
        ],
    )(h, W_rel)


def _sc_msg(tl, tr, gidx2, dst2, N, H):
    NCHUNKS = dst2.shape[0]
    TPR = NCHUNKS // NS
    NIO = 10
    NPT = N // NIO
    TPRH = TPR // 2

    @functools.partial(
        pl.kernel,
        out_type=jax.ShapeDtypeStruct((NC, N, H), f32),
        mesh=_mesh(),
        scratch_types=[
            pltpu.VMEM((TPRH, CHUNK), i32),
            pltpu.VMEM((TPRH, CHUNK), i32),
            pltpu.VMEM((CHUNK, H), f32),
            pltpu.SemaphoreType.DMA,
            pltpu.SemaphoreType.DMA,
            pltpu.VMEM_SHARED((N, H), f32),
        ],
    )
    def msg_kernel(tl_hbm, tr_hbm, gidx_hbm, dst_hbm, acc_hbm,
                   gbuf, dbuf, rows, gsem, ssem, shared):
        c = lax.axis_index("c")
        s = lax.axis_index("s")

        def zfill(k, _):
            rows[k // (H // L), pl.ds((k % (H // L)) * L, L)] = \
                jnp.zeros((L,), f32)
            return 0

        lax.fori_loop(0, CHUNK * (H // L), zfill, 0)

        @pl.when(s < NIO)
        def _():
            def zcopy(k, _):
                pltpu.sync_copy(rows.at[pl.ds(0, ZROWS)],
                                shared.at[pl.ds(s * NPT + k * ZROWS, ZROWS)])
                return 0

            lax.fori_loop(0, NPT // ZROWS, zcopy, 0)

        plsc.subcore_barrier()

        def run(t_ref):
            def body(i, _):
                pltpu.async_copy(t_ref.at[gbuf.at[i]], rows, gsem).wait()
                pltpu.async_copy(rows, shared.at[dbuf.at[i]], ssem,
                                 add=True).wait()
                return 0

            for p in range(TPR // TPRH):
                pltpu.sync_copy(
                    gidx_hbm.at[pl.ds(s * TPR + p * TPRH, TPRH)], gbuf)
                pltpu.sync_copy(
                    dst_hbm.at[pl.ds(s * TPR + p * TPRH, TPRH)], dbuf)
                lax.fori_loop(0, TPRH, body, 0)

        @pl.when(c == 0)
        def _():
            run(tl_hbm)

        @pl.when(c == 1)
        def _():
            run(tr_hbm)

        plsc.subcore_barrier()

        @pl.when(s < NIO)
        def _():
            pltpu.sync_copy(shared.at[pl.ds(s * NPT, NPT)],
                            acc_hbm.at[c, pl.ds(s * NPT, NPT)])

    return msg_kernel(tl, tr, gidx2, dst2)


def _tc_deg(d_a, d_b, NA, NB):
    E = d_a.shape[0]
    BE = 1000
    NG = E // BE

    def body(da_ref, db_ref, out_ref, acc):
        g = pl.program_id(0)

        @pl.when(g == 0)
        def _():
            acc[...] = jnp.zeros((NA, NB), f32)

        ia = lax.broadcasted_iota(i32, (BE, NA), 1)
        ib = lax.broadcasted_iota(i32, (BE, NB), 1)
        oa = (da_ref[...] == ia).astype(jnp.bfloat16)
        ob = (db_ref[...] == ib).astype(jnp.bfloat16)
        acc[...] += lax.dot_general(oa, ob, (((0,), (0,)), ((), ())),
                                    preferred_element_type=f32)

        @pl.when(g == NG - 1)
        def _():
            out_ref[...] = acc[...]

    return pl.pallas_call(
        body,
        grid=(NG,),
        in_specs=[
            pl.BlockSpec((BE, 1), lambda g: (g, 0)),
            pl.BlockSpec((BE, 1), lambda g: (g, 0)),
        ],
        out_specs=pl.BlockSpec((NA, NB), lambda g: (0, 0)),
        out_shape=jax.ShapeDtypeStruct((NA, NB), f32),
        scratch_shapes=[pltpu.VMEM((NA, NB), f32)],
    )(d_a, d_b)


def _tc_finish_transform(acc, deg3, h_in, W_self, b2d, W_rel_next, relu):
    N, D = h_in.shape
    R = W_rel_next.shape[0]
    H = D // 2
    NBN = N // BN

    def body(acc_ref, deg_ref, h_ref, ws_ref, b_ref, wr_ref,
             hout_ref, tl_ref, tr_ref, hscr):
        r = pl.program_id(1)

        @pl.when(r == 0)
        def _():
            inv = 1.0 / jnp.clip(deg_ref[...], 1.0, None)
            msg = jnp.concatenate([acc_ref[0] * inv, acc_ref[1] * inv], axis=1)
            base = jnp.dot(h_ref[...], ws_ref[...],
                           preferred_element_type=f32) + b_ref[...]
            hn = msg + base
            if relu:
                hn = jnp.maximum(hn, 0.0)
            hscr[...] = hn
            hout_ref[...] = hn

        res = jnp.dot(hscr[...], wr_ref[0], preferred_element_type=f32)
        tl_ref[...] = res[:, :H]
        tr_ref[...] = res[:, H:]

    return pl.pallas_call(
        body,
        grid=(NBN, R),
        in_specs=[
            pl.BlockSpec((NC, BN, H), lambda n, r: (0, n, 0)),
            pl.BlockSpec((BN, 1), lambda n, r: (n, 0)),
            pl.BlockSpec((BN, D), lambda n, r: (n, 0)),
            pl.BlockSpec((D, D), lambda n, r: (0, 0)),
            pl.BlockSpec((1, D), lambda n, r: (0, 0)),
            pl.BlockSpec((1, D, D), lambda n, r: (r, 0, 0)),
        ],
        out_specs=[
            pl.BlockSpec((BN, D), lambda n, r: (n, 0)),
            pl.BlockSpec((BN, H), lambda n, r: (r * NBN + n, 0)),
            pl.BlockSpec((BN, H), lambda n, r: (r * NBN + n, 0)),
        ],
        out_shape=[
            jax.ShapeDtypeStruct((N, D), f32),
            jax.ShapeDtypeStruct((R * N, H), f32),
            jax.ShapeDtypeStruct((R * N, H), f32),
        ],
        scratch_shapes=[pltpu.VMEM((BN, D), f32)],
    )(acc, deg3, h_in, W_self, b2d, W_rel_next)


def _tc_finish(acc, deg3, h_in, W_self, b2d):
    N, D = h_in.shape
    H = D // 2
    NBN = N // BN

    def body(acc_ref, deg_ref, h_ref, ws_ref, b_ref, out_ref):
        inv = 1.0 / jnp.clip(deg_ref[...], 1.0, None)
        msg = jnp.concatenate([acc_ref[0] * inv, acc_ref[1] * inv], axis=1)
        base = jnp.dot(h_ref[...], ws_ref[...],
                       preferred_element_type=f32) + b_ref[...]
        out_ref[...] = msg + base

    return pl.pallas_call(
        body,
        grid=(NBN,),
        in_specs=[
            pl.BlockSpec((NC, BN, H), lambda n: (0, n, 0)),
            pl.BlockSpec((BN, 1), lambda n: (n, 0)),
            pl.BlockSpec((BN, D), lambda n: (n, 0)),
            pl.BlockSpec((D, D), lambda n: (0, 0)),
            pl.BlockSpec((1, D), lambda n: (0, 0)),
        ],
        out_specs=pl.BlockSpec((BN, D), lambda n: (n, 0)),
        out_shape=jax.ShapeDtypeStruct((N, D), f32),
    )(acc, deg3, h_in, W_self, b2d)


def kernel(edges, entity_embed_init, W_rel1, W_self1, b1, W_rel2, W_self2, b2):
    h = entity_embed_init
    N, D = h.shape
    R = W_rel1.shape[0]
    H = D // 2
    b1_2 = b1.reshape(1, D)
    b2_2 = b2.reshape(1, D)

    for i in range(edges.shape[0]):
        src = edges[i, 0].astype(i32)
        rel = (edges[i, 1] % R).astype(i32)
        dst = edges[i, 2].astype(i32)

        gidx2 = (rel * N + src).reshape(-1, CHUNK)
        dst2 = dst.reshape(-1, CHUNK)
        NB = 128
        NA = (N + NB - 1) // NB
        d_a = (dst // NB).reshape(-1, 1)
        d_b = (dst % NB).reshape(-1, 1)

        deg2d = _tc_deg(d_a, d_b, NA, NB)
        deg3 = deg2d.reshape(-1)[:N].reshape(N, 1)
        tl1, tr1 = _tc_transform(h, W_rel1)
        acc1 = _sc_msg(tl1, tr1, gidx2, dst2, N, H)
        h1, tl2, tr2 = _tc_finish_transform(acc1, deg3, h, W_self1, b1_2,
                                            W_rel2, relu=True)
        acc2 = _sc_msg(tl2, tr2, gidx2, dst2, N, H)
        h = _tc_finish(acc2, deg3, h1, W_self2, b2_2)
    return h


# speedup vs baseline: 3.4052x; 1.0010x over previous
import functools

import jax
import jax.numpy as jnp
from jax import lax
from jax.experimental import pallas as pl
from jax.experimental.pallas import tpu as pltpu
from jax.experimental.pallas import tpu_sc as plsc

f32 = jnp.float32
i32 = jnp.int32

NC = 2
NS = 16
L = 16

CHUNK = 125
BN = 1000
ZROWS = 40

MM_PREC = jax.lax.Precision.DEFAULT


def _mesh():
    return plsc.VectorSubcoreMesh(core_axis_name="c", subcore_axis_name="s")


def _tc_transform(h, W_rel):
    N, D = h.shape
    R = W_rel.shape[0]
    H = D // 2
    NBN = N // BN

    def body(h_ref, w_ref, outl_ref, outr_ref):
        res = jnp.dot(h_ref[...], w_ref[0], preferred_element_type=f32,
                      precision=MM_PREC)
        outl_ref[...] = res[:, :H]
        outr_ref[...] = res[:, H:]

    return pl.pallas_call(
        body,
        grid=(NBN, R),
        in_specs=[
            pl.BlockSpec((BN, D), lambda n, r: (n, 0)),
            pl.BlockSpec((1, D, D), lambda n, r: (r, 0, 0)),
        ],
        out_specs=[
            pl.BlockSpec((BN, H), lambda n, r: (r * NBN + n, 0)),
            pl.BlockSpec((BN, H), lambda n, r: (r * NBN + n, 0)),
        ],
        out_shape=[
            jax.ShapeDtypeStruct((R * N, H), f32),
            jax.ShapeDtypeStruct((R * N, H), f32),
        ],
    )(h, W_rel)


def _sc_msg(tl, tr, gidx2, dst2, N, H):
    NCHUNKS = dst2.shape[0]
    TPR = NCHUNKS // NS
    NIO = 10
    NPT = N // NIO
    TPRH = TPR // 2

    @functools.partial(
        pl.kernel,
        out_type=jax.ShapeDtypeStruct((NC, N, H), f32),
        mesh=_mesh(),
        scratch_types=[
            pltpu.VMEM((TPRH, CHUNK), i32),
            pltpu.VMEM((TPRH, CHUNK), i32),
            pltpu.VMEM((CHUNK, H), f32),
            pltpu.SemaphoreType.DMA,
            pltpu.SemaphoreType.DMA,
            pltpu.VMEM_SHARED((N, H), f32),
        ],
    )
    def msg_kernel(tl_hbm, tr_hbm, gidx_hbm, dst_hbm, acc_hbm,
                   gbuf, dbuf, rows, gsem, ssem, shared):
        c = lax.axis_index("c")
        s = lax.axis_index("s")

        def zfill(k, _):
            rows[k // (H // L), pl.ds((k % (H // L)) * L, L)] = \
                jnp.zeros((L,), f32)
            return 0

        lax.fori_loop(0, CHUNK * (H // L), zfill, 0)

        @pl.when(s < NIO)
        def _():
            def zcopy(k, _):
                pltpu.sync_copy(rows.at[pl.ds(0, ZROWS)],
                                shared.at[pl.ds(s * NPT + k * ZROWS, ZROWS)])
                return 0

            lax.fori_loop(0, NPT // ZROWS, zcopy, 0)

        plsc.subcore_barrier()

        def run(t_ref):
            def body(i, _):
                pltpu.async_copy(t_ref.at[gbuf.at[i]], rows, gsem).wait()
                pltpu.async_copy(rows, shared.at[dbuf.at[i]], ssem,
                                 add=True).wait()
                return 0

            for p in range(TPR // TPRH):
                pltpu.sync_copy(
                    gidx_hbm.at[pl.ds(s * TPR + p * TPRH, TPRH)], gbuf)
                pltpu.sync_copy(
                    dst_hbm.at[pl.ds(s * TPR + p * TPRH, TPRH)], dbuf)
                lax.fori_loop(0, TPRH, body, 0)

        @pl.when(c == 0)
        def _():
            run(tl_hbm)

        @pl.when(c == 1)
        def _():
            run(tr_hbm)

        plsc.subcore_barrier()

        @pl.when(s < NIO)
        def _():
            pltpu.sync_copy(shared.at[pl.ds(s * NPT, NPT)],
                            acc_hbm.at[c, pl.ds(s * NPT, NPT)])

    return msg_kernel(tl, tr, gidx2, dst2)


def _tc_deg(d_a, d_b, NA, NB):
    E = d_a.shape[0]
    BE = 1000
    NG = E // BE

    def body(da_ref, db_ref, out_ref, acc):
        g = pl.program_id(0)

        @pl.when(g == 0)
        def _():
            acc[...] = jnp.zeros((NA, NB), f32)

        ia = lax.broadcasted_iota(i32, (BE, NA), 1)
        ib = lax.broadcasted_iota(i32, (BE, NB), 1)
        oa = (da_ref[...] == ia).astype(jnp.bfloat16)
        ob = (db_ref[...] == ib).astype(jnp.bfloat16)
        acc[...] += lax.dot_general(oa, ob, (((0,), (0,)), ((), ())),
                                    preferred_element_type=f32)

        @pl.when(g == NG - 1)
        def _():
            out_ref[...] = acc[...]

    return pl.pallas_call(
        body,
        grid=(NG,),
        in_specs=[
            pl.BlockSpec((BE, 1), lambda g: (g, 0)),
            pl.BlockSpec((BE, 1), lambda g: (g, 0)),
        ],
        out_specs=pl.BlockSpec((NA, NB), lambda g: (0, 0)),
        out_shape=jax.ShapeDtypeStruct((NA, NB), f32),
        scratch_shapes=[pltpu.VMEM((NA, NB), f32)],
    )(d_a, d_b)


def _tc_finish_transform(acc, deg3, h_in, W_self, b2d, W_rel_next, relu):
    N, D = h_in.shape
    R = W_rel_next.shape[0]
    H = D // 2
    NBN = N // BN

    def body(acc_ref, deg_ref, h_ref, ws_ref, b_ref, wr_ref,
             hout_ref, tl_ref, tr_ref, hscr):
        r = pl.program_id(1)

        @pl.when(r == 0)
        def _():
            inv = 1.0 / jnp.clip(deg_ref[...], 1.0, None)
            msg = jnp.concatenate([acc_ref[0] * inv, acc_ref[1] * inv], axis=1)
            base = jnp.dot(h_ref[...], ws_ref[...],
                           preferred_element_type=f32,
                           precision=MM_PREC) + b_ref[...]
            hn = msg + base
            if relu:
                hn = jnp.maximum(hn, 0.0)
            hscr[...] = hn
            hout_ref[...] = hn

        res = jnp.dot(hscr[...], wr_ref[0], preferred_element_type=f32,
                      precision=MM_PREC)
        tl_ref[...] = res[:, :H]
        tr_ref[...] = res[:, H:]

    return pl.pallas_call(
        body,
        grid=(NBN, R),
        in_specs=[
            pl.BlockSpec((NC, BN, H), lambda n, r: (0, n, 0)),
            pl.BlockSpec((BN, 1), lambda n, r: (n, 0)),
            pl.BlockSpec((BN, D), lambda n, r: (n, 0)),
            pl.BlockSpec((D, D), lambda n, r: (0, 0)),
            pl.BlockSpec((1, D), lambda n, r: (0, 0)),
            pl.BlockSpec((1, D, D), lambda n, r: (r, 0, 0)),
        ],
        out_specs=[
            pl.BlockSpec((BN, D), lambda n, r: (n, 0)),
            pl.BlockSpec((BN, H), lambda n, r: (r * NBN + n, 0)),
            pl.BlockSpec((BN, H), lambda n, r: (r * NBN + n, 0)),
        ],
        out_shape=[
            jax.ShapeDtypeStruct((N, D), f32),
            jax.ShapeDtypeStruct((R * N, H), f32),
            jax.ShapeDtypeStruct((R * N, H), f32),
        ],
        scratch_shapes=[pltpu.VMEM((BN, D), f32)],
    )(acc, deg3, h_in, W_self, b2d, W_rel_next)


def _tc_finish(acc, deg3, h_in, W_self, b2d):
    N, D = h_in.shape
    H = D // 2
    NBN = N // BN

    def body(acc_ref, deg_ref, h_ref, ws_ref, b_ref, out_ref):
        inv = 1.0 / jnp.clip(deg_ref[...], 1.0, None)
        msg = jnp.concatenate([acc_ref[0] * inv, acc_ref[1] * inv], axis=1)
        base = jnp.dot(h_ref[...], ws_ref[...],
                       preferred_element_type=f32,
                       precision=MM_PREC) + b_ref[...]
        out_ref[...] = msg + base

    return pl.pallas_call(
        body,
        grid=(NBN,),
        in_specs=[
            pl.BlockSpec((NC, BN, H), lambda n: (0, n, 0)),
            pl.BlockSpec((BN, 1), lambda n: (n, 0)),
            pl.BlockSpec((BN, D), lambda n: (n, 0)),
            pl.BlockSpec((D, D), lambda n: (0, 0)),
            pl.BlockSpec((1, D), lambda n: (0, 0)),
        ],
        out_specs=pl.BlockSpec((BN, D), lambda n: (n, 0)),
        out_shape=jax.ShapeDtypeStruct((N, D), f32),
    )(acc, deg3, h_in, W_self, b2d)


def kernel(edges, entity_embed_init, W_rel1, W_self1, b1, W_rel2, W_self2, b2):
    h = entity_embed_init
    N, D = h.shape
    R = W_rel1.shape[0]
    H = D // 2
    b1_2 = b1.reshape(1, D)
    b2_2 = b2.reshape(1, D)

    for i in range(edges.shape[0]):
        src = edges[i, 0].astype(i32)
        rel = (edges[i, 1] % R).astype(i32)
        dst = edges[i, 2].astype(i32)

        gidx2 = (rel * N + src).reshape(-1, CHUNK)
        dst2 = dst.reshape(-1, CHUNK)
        NB = 128
        NA = (N + NB - 1) // NB
        d_a = (dst // NB).reshape(-1, 1)
        d_b = (dst % NB).reshape(-1, 1)

        deg2d = _tc_deg(d_a, d_b, NA, NB)
        deg3 = deg2d.reshape(-1)[:N].reshape(N, 1)
        tl1, tr1 = _tc_transform(h, W_rel1)
        acc1 = _sc_msg(tl1, tr1, gidx2, dst2, N, H)
        h1, tl2, tr2 = _tc_finish_transform(acc1, deg3, h, W_self1, b1_2,
                                            W_rel2, relu=True)
        acc2 = _sc_msg(tl2, tr2, gidx2, dst2, N, H)
        h = _tc_finish(acc2, deg3, h1, W_self2, b2_2)
    return h
